# baseline (speedup 1.0000x reference)
import functools

import jax
import jax.numpy as jnp
from jax import lax
from jax.experimental import pallas as pl
from jax.experimental.pallas import tpu as pltpu
from jax.experimental.pallas import tpu_sc as plsc

_N = 4096
_E = 128
_B = 1024
_XMAX = 100.0
_ALPHA = 0.75
_NC = 2
_NS = 16
_NW = _NC * _NS
_BPW = _B // _NW
_L = 16


def _sc_body(in_idx_hbm, out_idx_hbm, co_hbm_tab, w_in_hbm, w_out_hbm,
             b_in_hbm, b_out_hbm,
             dot_hbm, co_hbm, s_hbm,
             in_idx_v, out_idx_v,
             rows_in_v, rows_out_v, blk_v, bin_v, bout_v,
             acc_v, dot_stage, co_stage, s_stage,
             sem0, sem1, sem2, sem3, semco):
    wid = lax.axis_index("s") * _NC + lax.axis_index("c")
    base = wid * _BPW
    cp0 = pltpu.async_copy(in_idx_hbm.at[pl.ds(base, _BPW)], in_idx_v, sem0)
    cp1 = pltpu.async_copy(out_idx_hbm.at[pl.ds(base, _BPW)], out_idx_v, sem1)
    cp4 = pltpu.async_copy(b_in_hbm, bin_v, sem2)
    cp5 = pltpu.async_copy(b_out_hbm, bout_v, sem3)
    cp0.wait()
    cp1.wait()
    g0 = pltpu.async_copy(w_in_hbm.at[in_idx_v], rows_in_v, sem0)
    g1 = pltpu.async_copy(w_out_hbm.at[out_idx_v], rows_out_v, sem1)
    iota = lax.iota(jnp.int32, _L)
    co_cps = []
    for c2 in range(_BPW // _L):
        rvec = in_idx_v[pl.ds(c2 * _L, _L)]
        cvec = out_idx_v[pl.ds(c2 * _L, _L)]
        for jj in range(_L):
            j = c2 * _L + jj
            r = rvec[jj]
            c = cvec[jj]
            co_cps.append(pltpu.async_copy(
                co_hbm_tab.at[r, pl.ds((c // 128) * 128, 128)],
                blk_v.at[j], semco))
    g0.wait()
    g1.wait()
    cp4.wait()
    cp5.wait()
    for c2 in range(_BPW // _L):
        for jj in range(_L):
            j = c2 * _L + jj
            acc = rows_in_v[j, pl.ds(0, _L)] * rows_out_v[j, pl.ds(0, _L)]
            for c in range(1, _E // _L):
                acc = acc + (rows_in_v[j, pl.ds(c * _L, _L)]
                             * rows_out_v[j, pl.ds(c * _L, _L)])
            acc_v[jj, :] = acc
        dotv = plsc.load_gather(acc_v, [iota, jnp.zeros((_L,), jnp.int32)])
        for c in range(1, _L):
            dotv = dotv + plsc.load_gather(
                acc_v, [iota, jnp.full((_L,), c, jnp.int32)])
        biv = plsc.load_gather(bin_v, [in_idx_v[pl.ds(c2 * _L, _L)]])
        bov = plsc.load_gather(bout_v, [out_idx_v[pl.ds(c2 * _L, _L)]])
        dot_stage[pl.ds(c2 * _L, _L)] = dotv
        s_stage[pl.ds(c2 * _L, _L)] = biv + bov
    for cp in co_cps:
        cp.wait()
    for c2 in range(_BPW // _L):
        cm = lax.rem(out_idx_v[pl.ds(c2 * _L, _L)], 128)
        cov = plsc.load_gather(blk_v, [iota + c2 * _L, cm]) + 1.0
        co_stage[pl.ds(c2 * _L, _L)] = cov
    o0 = pltpu.async_copy(dot_stage, dot_hbm.at[pl.ds(base, _BPW)], sem0)
    o1 = pltpu.async_copy(co_stage, co_hbm.at[pl.ds(base, _BPW)], sem1)
    o2 = pltpu.async_copy(s_stage, s_hbm.at[pl.ds(base, _BPW)], sem2)
    o0.wait()
    o1.wait()
    o2.wait()


_sc_gather = functools.partial(
    pl.kernel,
    _sc_body,
    out_type=[
        jax.ShapeDtypeStruct((_B,), jnp.float32),
        jax.ShapeDtypeStruct((_B,), jnp.float32),
        jax.ShapeDtypeStruct((_B,), jnp.float32),
    ],
    mesh=plsc.VectorSubcoreMesh(core_axis_name="c", subcore_axis_name="s"),
    compiler_params=pltpu.CompilerParams(needs_layout_passes=False, skip_device_barrier=True),
    scratch_types=[
        pltpu.VMEM((_BPW,), jnp.int32),
        pltpu.VMEM((_BPW,), jnp.int32),
        pltpu.VMEM((_BPW, _E), jnp.float32),
        pltpu.VMEM((_BPW, _E), jnp.float32),
        pltpu.VMEM((_BPW, _E), jnp.float32),
        pltpu.VMEM((_N,), jnp.float32),
        pltpu.VMEM((_N,), jnp.float32),
        pltpu.VMEM((_L, _L), jnp.float32),
        pltpu.VMEM((_BPW,), jnp.float32),
        pltpu.VMEM((_BPW,), jnp.float32),
        pltpu.VMEM((_BPW,), jnp.float32),
        pltpu.SemaphoreType.DMA,
        pltpu.SemaphoreType.DMA,
        pltpu.SemaphoreType.DMA,
        pltpu.SemaphoreType.DMA,
        pltpu.SemaphoreType.DMA,
    ],
)()


def _tc_body(dot_ref, co_ref, s_ref, out_ref):
    dot = dot_ref[...]
    co = co_ref[...]
    s = s_ref[...]
    logco = jnp.log(co)
    w = jnp.where(co > _XMAX, 1.0, jnp.power(co / _XMAX, _ALPHA))
    d = dot - logco
    s1 = jnp.sum(w * d * d)
    s2 = jnp.sum(w * d)
    s3 = jnp.sum(w)
    t1 = jnp.sum(s)
    t2 = jnp.sum(s * s)
    out_ref[0, 0] = _B * s1 + 2.0 * s2 * t1 + s3 * t2


def kernel(input, output, co_oc, W_in, b_in, W_out, b_out):
    in_idx = input.astype(jnp.int32)
    out_idx = output.astype(jnp.int32)
    dot, co1, s = _sc_gather(
        in_idx, out_idx, co_oc,
        W_in, W_out, b_in.reshape(_N), b_out.reshape(_N))
    loss = pl.pallas_call(
        _tc_body,
        out_shape=jax.ShapeDtypeStruct((1, 1), jnp.float32),
        out_specs=pl.BlockSpec(memory_space=pltpu.SMEM),
        compiler_params=pltpu.CompilerParams(skip_device_barrier=True),
    )(dot.reshape(8, 128), co1.reshape(8, 128), s.reshape(8, 128))
    return loss.reshape(())


# speedup vs baseline: 1.0055x; 1.0055x over previous
import functools

import jax
import jax.numpy as jnp
from jax import lax
from jax.experimental import pallas as pl
from jax.experimental.pallas import tpu as pltpu
from jax.experimental.pallas import tpu_sc as plsc

_N = 4096
_E = 128
_B = 1024
_XMAX = 100.0
_ALPHA = 0.75
_NC = 2
_NS = 16
_NW = _NC * _NS
_BPW = _B // _NW
_L = 16


def _sc_body(in_idx_hbm, out_idx_hbm, co_hbm_tab, w_in_hbm, w_out_hbm,
             b_in_hbm, b_out_hbm,
             dot_hbm, co_hbm, s_hbm,
             in_idx_v, out_idx_v,
             rows_in_v, rows_out_v, blk_v, bin_v, bout_v,
             acc_v, dot_stage, co_stage, s_stage,
             sem0, sem1, sem2, sem3, semco):
    wid = lax.axis_index("s") * _NC + lax.axis_index("c")
    base = wid * _BPW
    cp0 = pltpu.async_copy(in_idx_hbm.at[pl.ds(base, _BPW)], in_idx_v, sem0)
    cp1 = pltpu.async_copy(out_idx_hbm.at[pl.ds(base, _BPW)], out_idx_v, sem1)
    cp4 = pltpu.async_copy(b_in_hbm, bin_v, sem2)
    cp5 = pltpu.async_copy(b_out_hbm, bout_v, sem3)
    with jax.named_scope("idx_wait_done"):
        cp0.wait()
        cp1.wait()
    g0 = pltpu.async_copy(w_in_hbm.at[in_idx_v], rows_in_v, sem0)
    g1 = pltpu.async_copy(w_out_hbm.at[out_idx_v], rows_out_v, sem1)
    iota = lax.iota(jnp.int32, _L)
    co_cps = []
    for c2 in range(_BPW // _L):
        rvec = in_idx_v[pl.ds(c2 * _L, _L)]
        cvec = out_idx_v[pl.ds(c2 * _L, _L)]
        for jj in range(_L):
            j = c2 * _L + jj
            r = rvec[jj]
            c = cvec[jj]
            co_cps.append(pltpu.async_copy(
                co_hbm_tab.at[r, pl.ds((c // 128) * 128, 128)],
                blk_v.at[j], semco))
    with jax.named_scope("w_bias_wait"):
        g0.wait()
        g1.wait()
        cp4.wait()
        cp5.wait()
    for c2 in range(_BPW // _L):
        for jj in range(_L):
            j = c2 * _L + jj
            acc = rows_in_v[j, pl.ds(0, _L)] * rows_out_v[j, pl.ds(0, _L)]
            for c in range(1, _E // _L):
                acc = acc + (rows_in_v[j, pl.ds(c * _L, _L)]
                             * rows_out_v[j, pl.ds(c * _L, _L)])
            acc_v[jj, :] = acc
        dotv = plsc.load_gather(acc_v, [iota, jnp.zeros((_L,), jnp.int32)])
        for c in range(1, _L):
            dotv = dotv + plsc.load_gather(
                acc_v, [iota, jnp.full((_L,), c, jnp.int32)])
        biv = plsc.load_gather(bin_v, [in_idx_v[pl.ds(c2 * _L, _L)]])
        bov = plsc.load_gather(bout_v, [out_idx_v[pl.ds(c2 * _L, _L)]])
        dot_stage[pl.ds(c2 * _L, _L)] = dotv
        s_stage[pl.ds(c2 * _L, _L)] = biv + bov
    with jax.named_scope("co_drain"):
        for cp in co_cps:
            cp.wait()
    for c2 in range(_BPW // _L):
        cm = lax.rem(out_idx_v[pl.ds(c2 * _L, _L)], 128)
        cov = plsc.load_gather(blk_v, [iota + c2 * _L, cm]) + 1.0
        co_stage[pl.ds(c2 * _L, _L)] = cov
    with jax.named_scope("outcopy"):
        pass
    o0 = pltpu.async_copy(dot_stage, dot_hbm.at[pl.ds(base, _BPW)], sem0)
    o1 = pltpu.async_copy(co_stage, co_hbm.at[pl.ds(base, _BPW)], sem1)
    o2 = pltpu.async_copy(s_stage, s_hbm.at[pl.ds(base, _BPW)], sem2)
    o0.wait()
    o1.wait()
    o2.wait()


_sc_gather = functools.partial(
    pl.kernel,
    _sc_body,
    out_type=[
        jax.ShapeDtypeStruct((_B,), jnp.float32),
        jax.ShapeDtypeStruct((_B,), jnp.float32),
        jax.ShapeDtypeStruct((_B,), jnp.float32),
    ],
    mesh=plsc.VectorSubcoreMesh(core_axis_name="c", subcore_axis_name="s"),
    compiler_params=pltpu.CompilerParams(needs_layout_passes=False, skip_device_barrier=True),
    scratch_types=[
        pltpu.VMEM((_BPW,), jnp.int32),
        pltpu.VMEM((_BPW,), jnp.int32),
        pltpu.VMEM((_BPW, _E), jnp.float32),
        pltpu.VMEM((_BPW, _E), jnp.float32),
        pltpu.VMEM((_BPW, _E), jnp.float32),
        pltpu.VMEM((_N,), jnp.float32),
        pltpu.VMEM((_N,), jnp.float32),
        pltpu.VMEM((_L, _L), jnp.float32),
        pltpu.VMEM((_BPW,), jnp.float32),
        pltpu.VMEM((_BPW,), jnp.float32),
        pltpu.VMEM((_BPW,), jnp.float32),
        pltpu.SemaphoreType.DMA,
        pltpu.SemaphoreType.DMA,
        pltpu.SemaphoreType.DMA,
        pltpu.SemaphoreType.DMA,
        pltpu.SemaphoreType.DMA,
    ],
)()


def _tc_body(dot_ref, co_ref, s_ref, out_ref):
    dot = dot_ref[...]
    co = co_ref[...]
    s = s_ref[...]
    logco = jnp.log(co)
    w = jnp.where(co > _XMAX, 1.0, jnp.power(co / _XMAX, _ALPHA))
    d = dot - logco
    s1 = jnp.sum(w * d * d)
    s2 = jnp.sum(w * d)
    s3 = jnp.sum(w)
    t1 = jnp.sum(s)
    t2 = jnp.sum(s * s)
    out_ref[0, 0] = _B * s1 + 2.0 * s2 * t1 + s3 * t2


def kernel(input, output, co_oc, W_in, b_in, W_out, b_out):
    in_idx = input.astype(jnp.int32)
    out_idx = output.astype(jnp.int32)
    dot, co1, s = _sc_gather(
        in_idx, out_idx, co_oc,
        W_in, W_out, b_in.reshape(_N), b_out.reshape(_N))
    loss = pl.pallas_call(
        _tc_body,
        out_shape=jax.ShapeDtypeStruct((1, 1), jnp.float32),
        out_specs=pl.BlockSpec(memory_space=pltpu.SMEM),
        compiler_params=pltpu.CompilerParams(skip_device_barrier=True),
    )(dot.reshape(8, 128), co1.reshape(8, 128), s.reshape(8, 128))
    return loss.reshape(())


# speedup vs baseline: 1.0080x; 1.0024x over previous
import functools

import jax
import jax.numpy as jnp
from jax import lax
from jax.experimental import pallas as pl
from jax.experimental.pallas import tpu as pltpu
from jax.experimental.pallas import tpu_sc as plsc

_N = 4096
_E = 128
_B = 1024
_XMAX = 100.0
_ALPHA = 0.75
_NC = 2
_NS = 16
_NW = _NC * _NS
_BPW = _B // _NW
_L = 16


def _sc_body(in_idx_hbm, out_idx_hbm, co_hbm_tab, w_in_hbm, w_out_hbm,
             b_in_hbm, b_out_hbm,
             dot_hbm, co_hbm, s_hbm,
             in_idx_v, out_idx_v,
             rows_in_v, rows_out_v, blk_v, bin_v, bout_v,
             acc_v, stage_v,
             sem0, sem1, sem2, sem3, semco):
    wid = lax.axis_index("s") * _NC + lax.axis_index("c")
    base = wid * _BPW
    cp0 = pltpu.async_copy(in_idx_hbm.at[pl.ds(base, _BPW)], in_idx_v, sem0)
    cp1 = pltpu.async_copy(out_idx_hbm.at[pl.ds(base, _BPW)], out_idx_v, sem1)
    cp4 = pltpu.async_copy(b_in_hbm, bin_v, sem2)
    cp5 = pltpu.async_copy(b_out_hbm, bout_v, sem3)
    with jax.named_scope("idx_wait_done"):
        cp0.wait()
        cp1.wait()
    ga0 = pltpu.async_copy(w_in_hbm.at[in_idx_v.at[pl.ds(0, _L)]],
                           rows_in_v.at[pl.ds(0, _L)], sem0)
    ga1 = pltpu.async_copy(w_out_hbm.at[out_idx_v.at[pl.ds(0, _L)]],
                           rows_out_v.at[pl.ds(0, _L)], sem1)
    gb0 = pltpu.async_copy(w_in_hbm.at[in_idx_v.at[pl.ds(_L, _L)]],
                           rows_in_v.at[pl.ds(_L, _L)], sem0)
    gb1 = pltpu.async_copy(w_out_hbm.at[out_idx_v.at[pl.ds(_L, _L)]],
                           rows_out_v.at[pl.ds(_L, _L)], sem1)
    iota = lax.iota(jnp.int32, _L)
    co_cps = []
    for c2 in range(_BPW // _L):
        rvec = in_idx_v[pl.ds(c2 * _L, _L)]
        cvec = out_idx_v[pl.ds(c2 * _L, _L)]
        for jj in range(_L):
            j = c2 * _L + jj
            r = rvec[jj]
            c = cvec[jj]
            co_cps.append(pltpu.async_copy(
                co_hbm_tab.at[r, pl.ds((c // 128) * 128, 128)],
                blk_v.at[j], semco))
    waits = [(ga0, ga1), (gb0, gb1)]
    for c2 in range(_BPW // _L):
        with jax.named_scope("w_wait"):
            waits[c2][0].wait()
            waits[c2][1].wait()
        for jj in range(_L):
            j = c2 * _L + jj
            acc = rows_in_v[j, pl.ds(0, _L)] * rows_out_v[j, pl.ds(0, _L)]
            for c in range(1, _E // _L):
                acc = acc + (rows_in_v[j, pl.ds(c * _L, _L)]
                             * rows_out_v[j, pl.ds(c * _L, _L)])
            acc_v[jj, :] = acc
        dotv = plsc.load_gather(acc_v, [iota, jnp.zeros((_L,), jnp.int32)])
        for c in range(1, _L):
            dotv = dotv + plsc.load_gather(
                acc_v, [iota, jnp.full((_L,), c, jnp.int32)])
        stage_v[0, pl.ds(c2 * _L, _L)] = dotv
    with jax.named_scope("bias_wait"):
        cp4.wait()
        cp5.wait()
    for c2 in range(_BPW // _L):
        biv = plsc.load_gather(bin_v, [in_idx_v[pl.ds(c2 * _L, _L)]])
        bov = plsc.load_gather(bout_v, [out_idx_v[pl.ds(c2 * _L, _L)]])
        stage_v[2, pl.ds(c2 * _L, _L)] = biv + bov
    with jax.named_scope("co_drain"):
        for cp in co_cps:
            cp.wait()
    for c2 in range(_BPW // _L):
        cm = lax.rem(out_idx_v[pl.ds(c2 * _L, _L)], 128)
        cov = plsc.load_gather(blk_v, [iota + c2 * _L, cm]) + 1.0
        stage_v[1, pl.ds(c2 * _L, _L)] = cov
    with jax.named_scope("outcopy"):
        o0 = pltpu.async_copy(stage_v.at[0], dot_hbm.at[pl.ds(base, _BPW)], sem0)
        o1 = pltpu.async_copy(stage_v.at[1], co_hbm.at[pl.ds(base, _BPW)], sem1)
        o2 = pltpu.async_copy(stage_v.at[2], s_hbm.at[pl.ds(base, _BPW)], sem2)
        o0.wait()
        o1.wait()
        o2.wait()


_sc_gather = functools.partial(
    pl.kernel,
    _sc_body,
    out_type=[
        jax.ShapeDtypeStruct((_B,), jnp.float32),
        jax.ShapeDtypeStruct((_B,), jnp.float32),
        jax.ShapeDtypeStruct((_B,), jnp.float32),
    ],
    mesh=plsc.VectorSubcoreMesh(core_axis_name="c", subcore_axis_name="s"),
    compiler_params=pltpu.CompilerParams(needs_layout_passes=False, skip_device_barrier=True),
    scratch_types=[
        pltpu.VMEM((_BPW,), jnp.int32),
        pltpu.VMEM((_BPW,), jnp.int32),
        pltpu.VMEM((_BPW, _E), jnp.float32),
        pltpu.VMEM((_BPW, _E), jnp.float32),
        pltpu.VMEM((_BPW, _E), jnp.float32),
        pltpu.VMEM((_N,), jnp.float32),
        pltpu.VMEM((_N,), jnp.float32),
        pltpu.VMEM((_L, _L), jnp.float32),
        pltpu.VMEM((3, _BPW), jnp.float32),
        pltpu.SemaphoreType.DMA,
        pltpu.SemaphoreType.DMA,
        pltpu.SemaphoreType.DMA,
        pltpu.SemaphoreType.DMA,
        pltpu.SemaphoreType.DMA,
    ],
)()


def _tc_body(dot_ref, co_ref, s_ref, out_ref):
    dot = dot_ref[...]
    co = co_ref[...]
    s = s_ref[...]
    logco = jnp.log(co)
    w = jnp.where(co > _XMAX, 1.0, jnp.power(co / _XMAX, _ALPHA))
    d = dot - logco
    s1 = jnp.sum(w * d * d)
    s2 = jnp.sum(w * d)
    s3 = jnp.sum(w)
    t1 = jnp.sum(s)
    t2 = jnp.sum(s * s)
    out_ref[0, 0] = _B * s1 + 2.0 * s2 * t1 + s3 * t2


def kernel(input, output, co_oc, W_in, b_in, W_out, b_out):
    in_idx = input.astype(jnp.int32)
    out_idx = output.astype(jnp.int32)
    dot, co1, sv = _sc_gather(
        in_idx, out_idx, co_oc,
        W_in, W_out, b_in.reshape(_N), b_out.reshape(_N))
    loss = pl.pallas_call(
        _tc_body,
        out_shape=jax.ShapeDtypeStruct((1, 1), jnp.float32),
        out_specs=pl.BlockSpec(memory_space=pltpu.SMEM),
        compiler_params=pltpu.CompilerParams(skip_device_barrier=True),
    )(dot.reshape(8, 128), co1.reshape(8, 128), sv.reshape(8, 128))
    return loss.reshape(())
